# baseline (speedup 1.0000x reference)
import functools

import jax
import jax.numpy as jnp
from jax import lax
from jax.experimental import pallas as pl
from jax.experimental.pallas import tpu as pltpu
from jax.experimental.pallas import tpu_sc as plsc

B = 4096
D = 1024
ED = 8
VOCAB = 100
SORT = 15
BLK = 1024

NC, NS, L = 2, 16, 16
NW = NC * NS
RW = B // NW


def _dot(a, b):
    return jax.lax.dot_general(
        a, b, (((1,), (0,)), ((), ())),
        preferred_element_type=jnp.float32)


def _body_a(xs_ref, noise_ref, table_ref, w_sort_ref, b_sort_ref,
            prob_ref, scores_ref, ws_bf):
    @pl.when(pl.program_id(0) == 0)
    def _init():
        ws_bf[...] = w_sort_ref[...].astype(jnp.bfloat16)

    xs = xs_ref[...].astype(jnp.bfloat16)
    parent = table_ref[1:2, :].astype(jnp.bfloat16)

    const_sort = _dot(parent, ws_bf[:ED, :]) + b_sort_ref[...]
    logits = _dot(xs, ws_bf[2 * ED:, :]) + const_sort

    m = jnp.max(logits, axis=-1, keepdims=True)
    e = jnp.exp(logits - m)
    p = e / jnp.sum(e, axis=-1, keepdims=True)
    prob_ref[...] = p

    gumbel = -jnp.log(-jnp.log(noise_ref[...]))
    scores = jnp.log(p) + gumbel
    scores_ref[...] = scores.T


def _body_sc(scores_hbm, sort_hbm, scores_v, sort_v):
    wid = lax.axis_index("s") * NC + lax.axis_index("c")
    base = wid * RW
    for j in range(SORT):
        pltpu.sync_copy(scores_hbm.at[pl.ds(j * B + base, RW)],
                        scores_v.at[pl.ds(j * RW, RW)])

    def group(c, carry):
        off = pl.multiple_of(c * L, L)
        m = scores_v[pl.ds(off, L)]
        am = jnp.zeros((L,), jnp.int32)
        for j in range(1, SORT):
            v = scores_v[pl.ds(j * RW + off, L)]
            upd = v > m
            am = jnp.where(upd, j, am)
            m = jnp.where(upd, v, m)
        sort_v[pl.ds(off, L)] = am
        return carry

    lax.fori_loop(0, RW // L, group, 0)
    pltpu.sync_copy(sort_v, sort_hbm.at[pl.ds(base, RW)])


def _body_c(xs_ref, sort_ref, table_ref, w_note_ref, b_note_ref,
            note_ref, wn_bf):
    @pl.when(pl.program_id(0) == 0)
    def _init():
        wn_bf[...] = w_note_ref[...].astype(jnp.bfloat16)

    xs = xs_ref[...].astype(jnp.bfloat16)
    parent = table_ref[1:2, :].astype(jnp.bfloat16)

    idx = sort_ref[...]
    onehot = (jax.lax.broadcasted_iota(jnp.int32, (BLK, VOCAB), 1)
              == idx).astype(jnp.bfloat16)
    emb = _dot(onehot, table_ref[...].astype(jnp.bfloat16))
    emb = emb.astype(jnp.bfloat16)

    xe = jnp.concatenate([xs, emb], axis=1)
    const_note = _dot(parent, wn_bf[:ED, :]) + b_note_ref[...]
    note_ref[...] = _dot(xe, wn_bf[2 * ED:, :]) + const_note

def kernel(xs, noise, table, W_mask, b_mask, W_sort, b_sort, W_note, b_note):
    del W_mask, b_mask
    grid = (B // BLK,)

    prob, scores = pl.pallas_call(
        _body_a,
        grid=grid,
        in_specs=[
            pl.BlockSpec((BLK, D), lambda i: (i, 0)),
            pl.BlockSpec((BLK, SORT), lambda i: (i, 0)),
            pl.BlockSpec((VOCAB, ED), lambda i: (0, 0)),
            pl.BlockSpec((2 * ED + D, SORT), lambda i: (0, 0)),
            pl.BlockSpec((1, SORT), lambda i: (0, 0)),
        ],
        out_specs=[
            pl.BlockSpec((BLK, SORT), lambda i: (i, 0)),
            pl.BlockSpec((SORT, BLK), lambda i: (0, i)),
        ],
        out_shape=[
            jax.ShapeDtypeStruct((B, SORT), jnp.float32),
            jax.ShapeDtypeStruct((SORT, B), jnp.float32),
        ],
        scratch_shapes=[pltpu.VMEM((2 * ED + D, SORT), jnp.bfloat16)],
        compiler_params=pltpu.CompilerParams(
            dimension_semantics=("arbitrary",)),
    )(xs, noise, table, W_sort, b_sort.reshape(1, SORT))

    sc = pl.kernel(
        _body_sc,
        out_type=jax.ShapeDtypeStruct((B,), jnp.int32),
        mesh=plsc.VectorSubcoreMesh(core_axis_name="c", subcore_axis_name="s"),
        scratch_types=[
            pltpu.VMEM((SORT * RW,), jnp.float32),
            pltpu.VMEM((RW,), jnp.int32),
        ],
    )
    sort = sc(scores.reshape(SORT * B))

    note = pl.pallas_call(
        _body_c,
        grid=grid,
        in_specs=[
            pl.BlockSpec((BLK, D), lambda i: (i, 0)),
            pl.BlockSpec((BLK, 1), lambda i: (i, 0)),
            pl.BlockSpec((VOCAB, ED), lambda i: (0, 0)),
            pl.BlockSpec((2 * ED + D + ED, D), lambda i: (0, 0)),
            pl.BlockSpec((1, D), lambda i: (0, 0)),
        ],
        out_specs=pl.BlockSpec((BLK, D), lambda i: (i, 0)),
        out_shape=jax.ShapeDtypeStruct((B, D), jnp.float32),
        scratch_shapes=[pltpu.VMEM((2 * ED + D + ED, D), jnp.bfloat16)],
        compiler_params=pltpu.CompilerParams(
            dimension_semantics=("arbitrary",)),
    )(xs, sort.reshape(B, 1), table, W_note, b_note.reshape(1, D))

    return note, sort, prob


# speedup vs baseline: 1.0001x; 1.0001x over previous
import jax
import jax.numpy as jnp
from jax import lax
from jax.experimental import pallas as pl
from jax.experimental.pallas import tpu as pltpu
from jax.experimental.pallas import tpu_sc as plsc

B = 4096
D = 1024
ED = 8
VOCAB = 100
SORT = 15
BLK = 1024

NC, NS, L = 2, 16, 16
NW = NC * NS
RW = B // NW


def _dot(a, b):
    return jax.lax.dot_general(
        a, b, (((1,), (0,)), ((), ())),
        preferred_element_type=jnp.float32)


def _body_a(xs_ref, noise_ref, table_ref, w_sort_ref, b_sort_ref,
            prob_ref, scores_ref, ws_bf):
    @pl.when(pl.program_id(0) == 0)
    def _init():
        ws_bf[...] = w_sort_ref[...].astype(jnp.bfloat16)

    xs = xs_ref[...].astype(jnp.bfloat16)
    parent = table_ref[1:2, :].astype(jnp.bfloat16)

    const_sort = _dot(parent, ws_bf[:ED, :]) + b_sort_ref[...]
    logits = _dot(xs, ws_bf[2 * ED:, :]) + const_sort

    m = jnp.max(logits, axis=-1, keepdims=True)
    e = jnp.exp(logits - m)
    p = e / jnp.sum(e, axis=-1, keepdims=True)
    prob_ref[...] = p

    gumbel = -jnp.log(-jnp.log(noise_ref[...]))
    scores = jnp.log(p) + gumbel
    scores_ref[...] = scores.T


def _body_sc(scores_hbm, sort_hbm, scores_v, sort_v):
    wid = lax.axis_index("s") * NC + lax.axis_index("c")
    base = wid * RW
    for j in range(SORT):
        pltpu.sync_copy(scores_hbm.at[pl.ds(j * B + base, RW)],
                        scores_v.at[pl.ds(j * RW, RW)])

    def group(c, carry):
        off = pl.multiple_of(c * L, L)
        m = scores_v[pl.ds(off, L)]
        am = jnp.zeros((L,), jnp.int32)
        for j in range(1, SORT):
            v = scores_v[pl.ds(j * RW + off, L)]
            upd = v > m
            am = jnp.where(upd, j, am)
            m = jnp.where(upd, v, m)
        sort_v[pl.ds(off, L)] = am
        return carry

    lax.fori_loop(0, RW // L, group, 0)
    pltpu.sync_copy(sort_v, sort_hbm.at[pl.ds(base, RW)])


def _body_c(xs_ref, sort_ref, table_ref, w_note_ref, b_note_ref,
            note_ref, wn_bf):
    @pl.when(pl.program_id(0) == 0)
    def _init():
        wn_bf[...] = w_note_ref[...].astype(jnp.bfloat16)

    xs = xs_ref[...].astype(jnp.bfloat16)
    parent = table_ref[1:2, :].astype(jnp.bfloat16)

    idx = sort_ref[...]
    onehot = (jax.lax.broadcasted_iota(jnp.int32, (BLK, VOCAB), 1)
              == idx).astype(jnp.bfloat16)
    emb = _dot(onehot, table_ref[...].astype(jnp.bfloat16))
    emb = emb.astype(jnp.bfloat16)

    xe = jnp.concatenate([xs, emb], axis=1)
    const_note = _dot(parent, wn_bf[:ED, :]) + b_note_ref[...]
    note_ref[...] = _dot(xe, wn_bf[2 * ED:, :]) + const_note

def kernel(xs, noise, table, W_mask, b_mask, W_sort, b_sort, W_note, b_note):
    del W_mask, b_mask
    grid = (B // BLK,)

    prob, scores = pl.pallas_call(
        _body_a,
        grid=grid,
        in_specs=[
            pl.BlockSpec((BLK, D), lambda i: (i, 0)),
            pl.BlockSpec((BLK, SORT), lambda i: (i, 0)),
            pl.BlockSpec((VOCAB, ED), lambda i: (0, 0)),
            pl.BlockSpec((2 * ED + D, SORT), lambda i: (0, 0)),
            pl.BlockSpec((1, SORT), lambda i: (0, 0)),
        ],
        out_specs=[
            pl.BlockSpec((BLK, SORT), lambda i: (i, 0)),
            pl.BlockSpec((SORT, BLK), lambda i: (0, i)),
        ],
        out_shape=[
            jax.ShapeDtypeStruct((B, SORT), jnp.float32),
            jax.ShapeDtypeStruct((SORT, B), jnp.float32),
        ],
        scratch_shapes=[pltpu.VMEM((2 * ED + D, SORT), jnp.bfloat16)],
        compiler_params=pltpu.CompilerParams(
            dimension_semantics=("arbitrary",)),
    )(xs, noise, table, W_sort, b_sort.reshape(1, SORT))

    sc = pl.kernel(
        _body_sc,
        out_type=jax.ShapeDtypeStruct((B,), jnp.int32),
        mesh=plsc.VectorSubcoreMesh(core_axis_name="c", subcore_axis_name="s"),
        scratch_types=[
            pltpu.VMEM((SORT * RW,), jnp.float32),
            pltpu.VMEM((RW,), jnp.int32),
        ],
    )
    sort = sc(scores.reshape(SORT * B))

    note = pl.pallas_call(
        _body_c,
        grid=grid,
        in_specs=[
            pl.BlockSpec((BLK, D), lambda i: (i, 0)),
            pl.BlockSpec((BLK, 1), lambda i: (i, 0)),
            pl.BlockSpec((VOCAB, ED), lambda i: (0, 0)),
            pl.BlockSpec((2 * ED + D + ED, D), lambda i: (0, 0)),
            pl.BlockSpec((1, D), lambda i: (0, 0)),
        ],
        out_specs=pl.BlockSpec((BLK, D), lambda i: (i, 0)),
        out_shape=jax.ShapeDtypeStruct((B, D), jnp.float32),
        scratch_shapes=[pltpu.VMEM((2 * ED + D + ED, D), jnp.bfloat16)],
        compiler_params=pltpu.CompilerParams(
            dimension_semantics=("arbitrary",)),
    )(xs, sort.reshape(B, 1), table, W_note, b_note.reshape(1, D))

    return note, sort, prob
